# baseline (speedup 1.0000x reference)
import functools

import jax
import jax.numpy as jnp
from jax.experimental import pallas as pl


def _hinge_body(xt_ref, y_ref, loss_ref, *, n_classes, bl):
    ch = 8
    y = y_ref[...]
    sub = jax.lax.broadcasted_iota(jnp.int32, (ch, bl), 0)
    d = y[None, :] - sub

    acc_y = jnp.zeros((ch, bl), jnp.float32)
    for k in range(n_classes // ch):
        xk = xt_ref[k * ch:(k + 1) * ch, :]
        acc_y = acc_y + jnp.where(d == ch * k, xk, 0.0)
    t = jnp.sum(acc_y, axis=0) - 1.0

    acc_s = jnp.zeros((ch, bl), jnp.float32)
    for k in range(n_classes // ch):
        xk = xt_ref[k * ch:(k + 1) * ch, :]
        acc_s = acc_s + jnp.maximum(xk - t[None, :], 0.0)
    s = jnp.sum(acc_s, axis=0)

    loss_ref[...] = (s - 1.0) * (1.0 / n_classes)


def kernel(output, y):
    b, c = output.shape
    y = y.astype(jnp.int32)
    xt = output.T
    bl = 1024
    grid = (b // bl,)
    body = functools.partial(_hinge_body, n_classes=c, bl=bl)
    return pl.pallas_call(
        body,
        grid=grid,
        in_specs=[
            pl.BlockSpec((c, bl), lambda i: (0, i)),
            pl.BlockSpec((bl,), lambda i: (i,)),
        ],
        out_specs=pl.BlockSpec((bl,), lambda i: (i,)),
        out_shape=jax.ShapeDtypeStruct((b,), jnp.float32),
    )(xt, y)


# speedup vs baseline: 1.1106x; 1.1106x over previous
import functools

import jax
import jax.numpy as jnp
from jax.experimental import pallas as pl


def _hinge_body(xt_ref, y_ref, loss_ref, *, n_classes, bl):
    ch = 8
    y = y_ref[...]
    sub = jax.lax.broadcasted_iota(jnp.int32, (ch, bl), 0)
    d = y[None, :] - sub

    acc_y = jnp.zeros((ch, bl), jnp.float32)
    for k in range(n_classes // ch):
        xk = xt_ref[k * ch:(k + 1) * ch, :]
        acc_y = acc_y + jnp.where(d == ch * k, xk, 0.0)
    t = jnp.sum(acc_y, axis=0) - 1.0

    acc_s = jnp.zeros((ch, bl), jnp.float32)
    for k in range(n_classes // ch):
        xk = xt_ref[k * ch:(k + 1) * ch, :]
        acc_s = acc_s + jnp.maximum(xk - t[None, :], 0.0)
    s = jnp.sum(acc_s, axis=0)

    loss_ref[...] = (s - 1.0) * (1.0 / n_classes)


def kernel(output, y):
    b, c = output.shape
    y = y.astype(jnp.int32)
    xt = output.T
    bl = 4096
    grid = (b // bl,)
    body = functools.partial(_hinge_body, n_classes=c, bl=bl)
    return pl.pallas_call(
        body,
        grid=grid,
        in_specs=[
            pl.BlockSpec((c, bl), lambda i: (0, i)),
            pl.BlockSpec((bl,), lambda i: (i,)),
        ],
        out_specs=pl.BlockSpec((bl,), lambda i: (i,)),
        out_shape=jax.ShapeDtypeStruct((b,), jnp.float32),
    )(xt, y)
